# baseline (speedup 1.0000x reference)
import functools

import jax
import jax.numpy as jnp
from jax import lax
from jax.experimental import pallas as pl
from jax.experimental.pallas import tpu as pltpu
from jax.experimental.pallas import tpu_sc as plsc

_BS = 4
_P = 4096
_K = 32
_R2 = 0.2 * 0.2
_L = 16
_NW = 32
_CPW = _BS * _P // _NW
_WPB = _P // _CPW
_NV = _P // _L
_CHUNK = 16
_SUB = 8
_NCH = _NV // _CHUNK
_SCRATCH = 64
_CAP = _SCRATCH - _L


def _radius_body(xyz_hbm, nbr_hbm, fil_hbm, x_ref, y_ref, z_ref,
                 sc_ref, nbr_v, fil_v, cnt_ref):
    wid = lax.axis_index("s") * 2 + lax.axis_index("c")
    b = wid // _WPB
    c0 = (wid % _WPB) * _CPW

    pltpu.sync_copy(xyz_hbm.at[pl.ds((b * 3 + 0) * _P, _P)],
                    x_ref.at[pl.ds(0, _P)])
    pltpu.sync_copy(xyz_hbm.at[pl.ds((b * 3 + 1) * _P, _P)],
                    y_ref.at[pl.ds(0, _P)])
    pltpu.sync_copy(xyz_hbm.at[pl.ds((b * 3 + 2) * _P, _P)],
                    z_ref.at[pl.ds(0, _P)])

    lanes = lax.iota(jnp.int32, 16)

    def per_pair(j, _):
        c = c0 + 2 * j
        vx = x_ref[pl.ds(c, _L)]
        vy = y_ref[pl.ds(c, _L)]
        vz = z_ref[pl.ds(c, _L)]
        cx0 = jnp.full((16,), vx[0], jnp.float32)
        cy0 = jnp.full((16,), vy[0], jnp.float32)
        cz0 = jnp.full((16,), vz[0], jnp.float32)
        cx1 = jnp.full((16,), vx[1], jnp.float32)
        cy1 = jnp.full((16,), vy[1], jnp.float32)
        cz1 = jnp.full((16,), vz[1], jnp.float32)

        cnt_ref[0] = 0
        cnt_ref[1] = 0

        def chunk_step(ch, _c):
            @pl.when((cnt_ref[0] < _K) | (cnt_ref[1] < _K))
            def _do_chunk():
                base = ch * (_CHUNK * _L)
                for g in range(0, _CHUNK, _SUB):
                    gbase = base + g * _L
                    m0s, m1s = [], []
                    for u in range(_SUB):
                        off = gbase + u * _L
                        px = x_ref[pl.ds(off, _L)]
                        py = y_ref[pl.ds(off, _L)]
                        pz = z_ref[pl.ds(off, _L)]
                        dx0 = px - cx0
                        dy0 = py - cy0
                        dz0 = pz - cz0
                        m0s.append(dx0 * dx0 + dy0 * dy0 + dz0 * dz0 < _R2)
                        dx1 = px - cx1
                        dy1 = py - cy1
                        dz1 = pz - cz1
                        m1s.append(dx1 * dx1 + dy1 * dy1 + dz1 * dz1 < _R2)
                    cnt0 = cnt_ref[0]
                    cnt1 = cnt_ref[1]
                    for u in range(_SUB):
                        idxv = gbase + u * _L + lanes
                        o0 = jnp.minimum(cnt0, _CAP)
                        plsc.store_compressed(sc_ref.at[pl.ds(o0, _L)],
                                              idxv, mask=m0s[u])
                        cnt0 = cnt0 + plsc.all_reduce_population_count(
                            m0s[u])[0]
                        o1 = _SCRATCH + jnp.minimum(cnt1, _CAP)
                        plsc.store_compressed(sc_ref.at[pl.ds(o1, _L)],
                                              idxv, mask=m1s[u])
                        cnt1 = cnt1 + plsc.all_reduce_population_count(
                            m1s[u])[0]
                    cnt_ref[0] = cnt0
                    cnt_ref[1] = cnt1

            return 0

        lax.fori_loop(0, _NCH, chunk_step, 0)

        for s in range(2):
            cnt = jnp.full((16,), cnt_ref[s], jnp.int32)
            sb = s * _SCRATCH
            v0 = sc_ref[pl.ds(sb, _L)]
            v1 = sc_ref[pl.ds(sb + _L, _L)]
            first = jnp.full((16,), v0[0], jnp.int32)
            f0 = lax.shift_right_logical(lanes - cnt, 31)
            f1 = lax.shift_right_logical((lanes + _L) - cnt, 31)
            o = (2 * j + s) * _K
            nbr_v[pl.ds(o, _L)] = v0 * f0 + first * (1 - f0)
            nbr_v[pl.ds(o + _L, _L)] = v1 * f1 + first * (1 - f1)
            fil_v[pl.ds(o, _L)] = f0
            fil_v[pl.ds(o + _L, _L)] = f1
        return 0

    lax.fori_loop(0, _CPW // 2, per_pair, 0)

    pltpu.sync_copy(nbr_v, nbr_hbm.at[pl.ds(wid * _CPW * _K, _CPW * _K)])
    pltpu.sync_copy(fil_v, fil_hbm.at[pl.ds(wid * _CPW * _K, _CPW * _K)])


_radius_sc = functools.partial(
    pl.kernel,
    mesh=plsc.VectorSubcoreMesh(core_axis_name="c", subcore_axis_name="s"),
    out_type=[
        jax.ShapeDtypeStruct((_BS * _P * _K,), jnp.int32),
        jax.ShapeDtypeStruct((_BS * _P * _K,), jnp.int32),
    ],
    scratch_types=[
        pltpu.VMEM((_P + _L,), jnp.float32),
        pltpu.VMEM((_P + _L,), jnp.float32),
        pltpu.VMEM((_P + _L,), jnp.float32),
        pltpu.VMEM((2 * _SCRATCH,), jnp.int32),
        pltpu.VMEM((_CPW * _K,), jnp.int32),
        pltpu.VMEM((_CPW * _K,), jnp.int32),
        pltpu.SMEM((2,), jnp.int32),
    ],
    compiler_params=pltpu.CompilerParams(needs_layout_passes=False),
)(_radius_body)


def kernel(xyz):
    bs, p = xyz.shape[:2]
    xyz_t = jnp.transpose(xyz, (0, 2, 1)).reshape(-1)
    nbr_f, fil_f = _radius_sc(xyz_t)
    nbr = nbr_f.reshape(bs, p, _K)
    filled = fil_f.reshape(bs, p, _K) != 0
    ctr = jnp.broadcast_to(
        jnp.arange(p, dtype=jnp.int32)[None, :, None], (bs, p, _K))
    edges = jnp.stack([nbr, ctr], axis=-1)
    return edges, filled, xyz
